# initial kernel scaffold (unmeasured)
import jax
import jax.numpy as jnp
from jax import lax
from jax.experimental import pallas as pl
from jax.experimental.pallas import tpu as pltpu

N_DEV = 32


def kernel(x, w_mat):
    m_per, k = x.shape
    k2, n_per = w_mat.shape
    assert k == k2

    def body(x_ref, w_ref, out_ref, comm_ref, send_sems, recv_sems):
        my = lax.axis_index("i")
        left = jnp.mod(my - 1, N_DEV)
        right = jnp.mod(my + 1, N_DEV)

        barrier_sem = pltpu.get_barrier_semaphore()
        for nbr in (left, right):
            pl.semaphore_signal(
                barrier_sem, inc=1,
                device_id=(nbr,), device_id_type=pl.DeviceIdType.MESH,
            )
        pl.semaphore_wait(barrier_sem, 2)

        comm_ref[my] = x_ref[:, :]
        out_ref[pl.ds(my * m_per, m_per), :] = jnp.dot(
            x_ref[:, :], w_ref[:, :], preferred_element_type=jnp.float32
        )

        sends = []
        for h in range(1, N_DEV):
            fwd_o = jnp.mod(my - (h - 1), N_DEV)
            send = pltpu.make_async_remote_copy(
                src_ref=comm_ref.at[fwd_o],
                dst_ref=comm_ref.at[fwd_o],
                send_sem=send_sems.at[h - 1],
                recv_sem=recv_sems.at[h - 1],
                device_id=(right,),
                device_id_type=pl.DeviceIdType.MESH,
            )
            send.start()
            sends.append(send)

            recv_o = jnp.mod(my - h, N_DEV)
            recv = pltpu.make_async_remote_copy(
                src_ref=comm_ref.at[recv_o],
                dst_ref=comm_ref.at[recv_o],
                send_sem=send_sems.at[h - 1],
                recv_sem=recv_sems.at[h - 1],
                device_id=(left,),
                device_id_type=pl.DeviceIdType.MESH,
            )
            recv.wait_recv()
            out_ref[pl.ds(recv_o * m_per, m_per), :] = jnp.dot(
                comm_ref[recv_o], w_ref[:, :], preferred_element_type=jnp.float32
            )

        for send in sends:
            send.wait_send()

    out_shape = jax.ShapeDtypeStruct((N_DEV * m_per, n_per), jnp.float32)
    return pl.pallas_call(
        body,
        out_shape=out_shape,
        in_specs=[
            pl.BlockSpec(memory_space=pltpu.VMEM),
            pl.BlockSpec(memory_space=pltpu.VMEM),
        ],
        out_specs=pl.BlockSpec(memory_space=pltpu.VMEM),
        scratch_shapes=[
            pltpu.VMEM((N_DEV, m_per, k), x.dtype),
            pltpu.SemaphoreType.DMA((N_DEV - 1,)),
            pltpu.SemaphoreType.DMA((N_DEV - 1,)),
        ],
        compiler_params=pltpu.CompilerParams(collective_id=0),
    )(x, w_mat)


# baseline (device time: 428530 ns/iter reference)
import jax
import jax.numpy as jnp
from jax import lax
from jax.experimental import pallas as pl
from jax.experimental.pallas import tpu as pltpu

N_DEV = 32


def kernel(x, w_mat):
    x = x.astype(jnp.bfloat16)
    w_mat = w_mat.astype(jnp.bfloat16)
    m_per, k = x.shape
    k2, n_per = w_mat.shape
    assert k == k2

    def body(x_ref, w_ref, out_ref, comm_ref, send_sems, recv_sems):
        my = lax.axis_index("i")
        left = jnp.mod(my - 1, N_DEV)
        right = jnp.mod(my + 1, N_DEV)

        barrier_sem = pltpu.get_barrier_semaphore()
        for nbr in (left, right):
            pl.semaphore_signal(
                barrier_sem, inc=1,
                device_id=(nbr,), device_id_type=pl.DeviceIdType.MESH,
            )
        pl.semaphore_wait(barrier_sem, 2)

        comm_ref[my] = x_ref[:, :]
        out_ref[pl.ds(my * m_per, m_per), :] = jnp.dot(
            x_ref[:, :], w_ref[:, :], preferred_element_type=jnp.float32
        )

        sends = []
        for h in range(1, N_DEV):
            fwd_o = jnp.mod(my - (h - 1), N_DEV)
            send = pltpu.make_async_remote_copy(
                src_ref=comm_ref.at[fwd_o],
                dst_ref=comm_ref.at[fwd_o],
                send_sem=send_sems.at[h - 1],
                recv_sem=recv_sems.at[h - 1],
                device_id=(right,),
                device_id_type=pl.DeviceIdType.MESH,
            )
            send.start()
            sends.append(send)

            recv_o = jnp.mod(my - h, N_DEV)
            recv = pltpu.make_async_remote_copy(
                src_ref=comm_ref.at[recv_o],
                dst_ref=comm_ref.at[recv_o],
                send_sem=send_sems.at[h - 1],
                recv_sem=recv_sems.at[h - 1],
                device_id=(left,),
                device_id_type=pl.DeviceIdType.MESH,
            )
            recv.wait_recv()
            out_ref[pl.ds(recv_o * m_per, m_per), :] = jnp.dot(
                comm_ref[recv_o], w_ref[:, :], preferred_element_type=jnp.float32
            )

        for send in sends:
            send.wait_send()

    out_shape = jax.ShapeDtypeStruct((N_DEV * m_per, n_per), jnp.float32)
    return pl.pallas_call(
        body,
        out_shape=out_shape,
        in_specs=[
            pl.BlockSpec(memory_space=pltpu.VMEM),
            pl.BlockSpec(memory_space=pltpu.VMEM),
        ],
        out_specs=pl.BlockSpec(memory_space=pltpu.VMEM),
        scratch_shapes=[
            pltpu.VMEM((N_DEV, m_per, k), x.dtype),
            pltpu.SemaphoreType.DMA((N_DEV - 1,)),
            pltpu.SemaphoreType.DMA((N_DEV - 1,)),
        ],
        compiler_params=pltpu.CompilerParams(collective_id=0),
    )(x, w_mat)


# device time: 363277 ns/iter; 1.1796x vs baseline; 1.1796x over previous
import jax
import jax.numpy as jnp
from jax import lax
from jax.experimental import pallas as pl
from jax.experimental.pallas import tpu as pltpu

N_DEV = 32
R_HOPS = N_DEV // 2
L_HOPS = N_DEV - 1 - R_HOPS


def kernel(x, w_mat):
    x = x.astype(jnp.bfloat16)
    w_mat = w_mat.astype(jnp.bfloat16)
    m_per, k = x.shape
    k2, n_per = w_mat.shape
    assert k == k2

    def body(x_ref, w_ref, out_ref, comm_ref,
             send_r, recv_r, send_l, recv_l):
        my = lax.axis_index("i")
        left = jnp.mod(my - 1, N_DEV)
        right = jnp.mod(my + 1, N_DEV)

        barrier_sem = pltpu.get_barrier_semaphore()
        for nbr in (left, right):
            pl.semaphore_signal(
                barrier_sem, inc=1,
                device_id=(nbr,), device_id_type=pl.DeviceIdType.MESH,
            )
        pl.semaphore_wait(barrier_sem, 2)

        comm_ref[my] = x_ref[:, :]

        sends = []

        def send(origin, dst_dev, sem_arr, rsem_arr, idx):
            rdma = pltpu.make_async_remote_copy(
                src_ref=comm_ref.at[origin],
                dst_ref=comm_ref.at[origin],
                send_sem=sem_arr.at[idx],
                recv_sem=rsem_arr.at[idx],
                device_id=(dst_dev,),
                device_id_type=pl.DeviceIdType.MESH,
            )
            rdma.start()
            sends.append(rdma)

        def wait_recv(origin, src_dev, sem_arr, rsem_arr, idx):
            rdma = pltpu.make_async_remote_copy(
                src_ref=comm_ref.at[origin],
                dst_ref=comm_ref.at[origin],
                send_sem=sem_arr.at[idx],
                recv_sem=rsem_arr.at[idx],
                device_id=(src_dev,),
                device_id_type=pl.DeviceIdType.MESH,
            )
            rdma.wait_recv()

        def gemm(origin):
            out_ref[pl.ds(origin * m_per, m_per), :] = jnp.dot(
                comm_ref[origin], w_ref[:, :],
                preferred_element_type=jnp.float32,
            )

        send(my, right, send_r, recv_r, 0)
        send(my, left, send_l, recv_l, 0)
        gemm(my)

        for h in range(1, R_HOPS + 1):
            r_o = jnp.mod(my - h, N_DEV)
            wait_recv(r_o, left, send_r, recv_r, h - 1)
            if h + 1 <= R_HOPS:
                send(r_o, right, send_r, recv_r, h)

            if h <= L_HOPS:
                l_o = jnp.mod(my + h, N_DEV)
                wait_recv(l_o, right, send_l, recv_l, h - 1)
                if h + 1 <= L_HOPS:
                    send(l_o, left, send_l, recv_l, h)
                gemm(r_o)
                gemm(l_o)
            else:
                gemm(r_o)

        for rdma in sends:
            rdma.wait_send()

    out_shape = jax.ShapeDtypeStruct((N_DEV * m_per, n_per), jnp.float32)
    return pl.pallas_call(
        body,
        out_shape=out_shape,
        in_specs=[
            pl.BlockSpec(memory_space=pltpu.VMEM),
            pl.BlockSpec(memory_space=pltpu.VMEM),
        ],
        out_specs=pl.BlockSpec(memory_space=pltpu.VMEM),
        scratch_shapes=[
            pltpu.VMEM((N_DEV, m_per, k), x.dtype),
            pltpu.SemaphoreType.DMA((R_HOPS,)),
            pltpu.SemaphoreType.DMA((R_HOPS,)),
            pltpu.SemaphoreType.DMA((L_HOPS,)),
            pltpu.SemaphoreType.DMA((L_HOPS,)),
        ],
        compiler_params=pltpu.CompilerParams(collective_id=0),
    )(x, w_mat)


# device time: 196808 ns/iter; 2.1774x vs baseline; 1.8458x over previous
import jax
import jax.numpy as jnp
from jax import lax
from jax.experimental import pallas as pl
from jax.experimental.pallas import tpu as pltpu

N_DEV = 32
R_HOPS = N_DEV // 2
L_HOPS = N_DEV - 1 - R_HOPS
N_SUB = 2

RING = [0, 1, 9, 8, 16, 17, 25, 24, 27, 26, 18, 19, 11, 10, 13, 12,
        20, 21, 29, 28, 31, 30, 22, 23, 15, 14, 6, 7, 4, 5, 2, 3]
POS = [0] * N_DEV
for _p, _m in enumerate(RING):
    POS[_m] = _p


def kernel(x, w_mat):
    x = x.astype(jnp.bfloat16)
    w_mat = w_mat.astype(jnp.bfloat16)
    m_per, k = x.shape
    k2, n_per = w_mat.shape
    assert k == k2
    m_half = m_per // N_SUB

    my = lax.axis_index("i")
    ring = jnp.array(RING, dtype=jnp.int32)
    pos = jnp.array(POS, dtype=jnp.int32)[my]
    right = ring[jnp.mod(pos + 1, N_DEV)]
    left = ring[jnp.mod(pos - 1, N_DEV)]
    r_orig = ring[jnp.mod(pos - jnp.arange(1, R_HOPS + 1), N_DEV)]
    l_orig = ring[jnp.mod(pos + jnp.arange(1, L_HOPS + 1), N_DEV)]
    meta = jnp.concatenate(
        [right[None], left[None], r_orig, l_orig]
    ).astype(jnp.int32)

    def body(meta_ref, x_ref, w_ref, out_ref, comm_ref,
             send_r, recv_r, send_l, recv_l):
        my_i = lax.axis_index("i")
        right_d = meta_ref[0]
        left_d = meta_ref[1]

        barrier_sem = pltpu.get_barrier_semaphore()
        for nbr in (left_d, right_d):
            pl.semaphore_signal(
                barrier_sem, inc=1,
                device_id=(nbr,), device_id_type=pl.DeviceIdType.MESH,
            )
        pl.semaphore_wait(barrier_sem, 2)

        for s in range(N_SUB):
            comm_ref[my_i * N_SUB + s] = x_ref[pl.ds(s * m_half, m_half), :]

        sends = []

        def make(origin, half, dev, sem_arr, rsem_arr, hop):
            slot = origin * N_SUB + half
            return pltpu.make_async_remote_copy(
                src_ref=comm_ref.at[slot],
                dst_ref=comm_ref.at[slot],
                send_sem=sem_arr.at[hop * N_SUB + half],
                recv_sem=rsem_arr.at[hop * N_SUB + half],
                device_id=(dev,),
                device_id_type=pl.DeviceIdType.MESH,
            )

        def send(origin, half, dev, sem_arr, rsem_arr, hop):
            rdma = make(origin, half, dev, sem_arr, rsem_arr, hop)
            rdma.start()
            sends.append(rdma)

        def gemm(origin):
            out_ref[pl.ds(origin * m_per, m_per), :] = jnp.dot(
                comm_ref[pl.ds(origin * N_SUB, N_SUB)].reshape(m_per, k),
                w_ref[:, :],
                preferred_element_type=jnp.float32,
            )

        for s in range(N_SUB):
            send(my_i, s, right_d, send_r, recv_r, 0)
            send(my_i, s, left_d, send_l, recv_l, 0)
        gemm(my_i)

        for h in range(1, R_HOPS + 1):
            r_o = meta_ref[2 + (h - 1)]
            do_l = h <= L_HOPS
            l_o = meta_ref[2 + R_HOPS + (h - 1)] if do_l else None
            for s in range(N_SUB):
                make(r_o, s, left_d, send_r, recv_r, h - 1).wait_recv()
                if h + 1 <= R_HOPS:
                    send(r_o, s, right_d, send_r, recv_r, h)
                if do_l:
                    make(l_o, s, right_d, send_l, recv_l, h - 1).wait_recv()
                    if h + 1 <= L_HOPS:
                        send(l_o, s, left_d, send_l, recv_l, h)
            gemm(r_o)
            if do_l:
                gemm(l_o)

        for rdma in sends:
            rdma.wait_send()

    out_shape = jax.ShapeDtypeStruct((N_DEV * m_per, n_per), jnp.float32)
    return pl.pallas_call(
        body,
        out_shape=out_shape,
        in_specs=[
            pl.BlockSpec(memory_space=pltpu.SMEM),
            pl.BlockSpec(memory_space=pltpu.VMEM),
            pl.BlockSpec(memory_space=pltpu.VMEM),
        ],
        out_specs=pl.BlockSpec(memory_space=pltpu.VMEM),
        scratch_shapes=[
            pltpu.VMEM((N_DEV * N_SUB, m_half, k), x.dtype),
            pltpu.SemaphoreType.DMA((R_HOPS * N_SUB,)),
            pltpu.SemaphoreType.DMA((R_HOPS * N_SUB,)),
            pltpu.SemaphoreType.DMA((L_HOPS * N_SUB,)),
            pltpu.SemaphoreType.DMA((L_HOPS * N_SUB,)),
        ],
        compiler_params=pltpu.CompilerParams(collective_id=0),
    )(meta, x, w_mat)


# device time: 192098 ns/iter; 2.2308x vs baseline; 1.0245x over previous
import jax
import jax.numpy as jnp
from jax import lax
from jax.experimental import pallas as pl
from jax.experimental.pallas import tpu as pltpu

N_DEV = 32
AP = N_DEV // 2
N_SUB = 2

RING = [0, 1, 9, 8, 16, 17, 25, 24, 27, 26, 18, 19, 11, 10, 13, 12,
        20, 21, 29, 28, 31, 30, 22, 23, 15, 14, 6, 7, 4, 5, 2, 3]
POS = [0] * N_DEV
for _p, _m in enumerate(RING):
    POS[_m] = _p


def kernel(x, w_mat):
    x = x.astype(jnp.bfloat16)
    w_mat = w_mat.astype(jnp.bfloat16)
    m_per, k = x.shape
    k2, n_per = w_mat.shape
    assert k == k2
    m_half = m_per // N_SUB

    my = lax.axis_index("i")
    ring = jnp.array(RING, dtype=jnp.int32)
    pos = jnp.array(POS, dtype=jnp.int32)[my]
    right = ring[jnp.mod(pos + 1, N_DEV)]
    left = ring[jnp.mod(pos - 1, N_DEV)]
    r_orig = ring[jnp.mod(pos - jnp.arange(1, AP + 1), N_DEV)]
    l_orig = ring[jnp.mod(pos + jnp.arange(1, AP + 1), N_DEV)]
    meta = jnp.concatenate(
        [right[None], left[None], r_orig, l_orig]
    ).astype(jnp.int32)

    def body(meta_ref, x_ref, w_ref, out_ref, comm_ref,
             send_r, recv_r, send_l, recv_l):
        my_i = lax.axis_index("i")
        right_d = meta_ref[0]
        left_d = meta_ref[1]

        barrier_sem = pltpu.get_barrier_semaphore()
        for nbr in (left_d, right_d):
            pl.semaphore_signal(
                barrier_sem, inc=1,
                device_id=(nbr,), device_id_type=pl.DeviceIdType.MESH,
            )
        pl.semaphore_wait(barrier_sem, 2)

        sends = []

        def send(src, origin, half, dev, sem_arr, rsem_arr, hop):
            rdma = pltpu.make_async_remote_copy(
                src_ref=src,
                dst_ref=comm_ref.at[origin * N_SUB + half],
                send_sem=sem_arr.at[hop * N_SUB + half],
                recv_sem=rsem_arr.at[hop * N_SUB + half],
                device_id=(dev,),
                device_id_type=pl.DeviceIdType.MESH,
            )
            rdma.start()
            sends.append(rdma)

        def wait_recv(origin, half, dev, sem_arr, rsem_arr, hop):
            slot = origin * N_SUB + half
            pltpu.make_async_remote_copy(
                src_ref=comm_ref.at[slot],
                dst_ref=comm_ref.at[slot],
                send_sem=sem_arr.at[hop * N_SUB + half],
                recv_sem=rsem_arr.at[hop * N_SUB + half],
                device_id=(dev,),
                device_id_type=pl.DeviceIdType.MESH,
            ).wait_recv()

        def gemm(origin):
            out_ref[pl.ds(origin * m_per, m_per), :] = jnp.dot(
                comm_ref[pl.ds(origin * N_SUB, N_SUB)].reshape(m_per, k),
                w_ref[:, :],
                preferred_element_type=jnp.float32,
            )

        for s in range(N_SUB):
            half_src = x_ref.at[pl.ds(s * m_half, m_half), :]
            send(half_src, my_i, s, right_d, send_r, recv_r, 0)
            send(half_src, my_i, s, left_d, send_l, recv_l, 0)
        out_ref[pl.ds(my_i * m_per, m_per), :] = jnp.dot(
            x_ref[:, :], w_ref[:, :], preferred_element_type=jnp.float32
        )

        for h in range(1, AP + 1):
            r_o = meta_ref[2 + (h - 1)]
            l_o = meta_ref[2 + AP + (h - 1)]
            for s in range(N_SUB):
                if h < AP or s == 0:
                    wait_recv(r_o, s, left_d, send_r, recv_r, h - 1)
                    if h + 1 < AP or (h + 1 == AP and s == 0):
                        send(comm_ref.at[r_o * N_SUB + s], r_o, s,
                             right_d, send_r, recv_r, h)
                if h < AP or s == 1:
                    wait_recv(l_o, s, right_d, send_l, recv_l, h - 1)
                    if h + 1 < AP or (h + 1 == AP and s == 1):
                        send(comm_ref.at[l_o * N_SUB + s], l_o, s,
                             left_d, send_l, recv_l, h)
            gemm(r_o)
            if h < AP:
                gemm(l_o)

        for rdma in sends:
            rdma.wait_send()

    out_shape = jax.ShapeDtypeStruct((N_DEV * m_per, n_per), jnp.float32)
    return pl.pallas_call(
        body,
        out_shape=out_shape,
        in_specs=[
            pl.BlockSpec(memory_space=pltpu.SMEM),
            pl.BlockSpec(memory_space=pltpu.VMEM),
            pl.BlockSpec(memory_space=pltpu.VMEM),
        ],
        out_specs=pl.BlockSpec(memory_space=pltpu.VMEM),
        scratch_shapes=[
            pltpu.VMEM((N_DEV * N_SUB, m_half, k), x.dtype),
            pltpu.SemaphoreType.DMA((AP * N_SUB,)),
            pltpu.SemaphoreType.DMA((AP * N_SUB,)),
            pltpu.SemaphoreType.DMA((AP * N_SUB,)),
            pltpu.SemaphoreType.DMA((AP * N_SUB,)),
        ],
        compiler_params=pltpu.CompilerParams(collective_id=0),
    )(meta, x, w_mat)


# device time: 191850 ns/iter; 2.2337x vs baseline; 1.0013x over previous
import jax
import jax.numpy as jnp
from jax import lax
from jax.experimental import pallas as pl
from jax.experimental.pallas import tpu as pltpu

N_DEV = 32
AP = N_DEV // 2
N_SUB = 4
HALF = N_SUB // 2

RING = [0, 1, 9, 8, 16, 17, 25, 24, 27, 26, 18, 19, 11, 10, 13, 12,
        20, 21, 29, 28, 31, 30, 22, 23, 15, 14, 6, 7, 4, 5, 2, 3]
POS = [0] * N_DEV
for _p, _m in enumerate(RING):
    POS[_m] = _p


def kernel(x, w_mat):
    x = x.astype(jnp.bfloat16)
    w_mat = w_mat.astype(jnp.bfloat16)
    m_per, k = x.shape
    k2, n_per = w_mat.shape
    assert k == k2
    m_half = m_per // N_SUB

    my = lax.axis_index("i")
    ring = jnp.array(RING, dtype=jnp.int32)
    pos = jnp.array(POS, dtype=jnp.int32)[my]
    right = ring[jnp.mod(pos + 1, N_DEV)]
    left = ring[jnp.mod(pos - 1, N_DEV)]
    r_orig = ring[jnp.mod(pos - jnp.arange(1, AP + 1), N_DEV)]
    l_orig = ring[jnp.mod(pos + jnp.arange(1, AP + 1), N_DEV)]
    meta = jnp.concatenate(
        [right[None], left[None], r_orig, l_orig]
    ).astype(jnp.int32)

    def body(meta_ref, x_ref, w_ref, out_ref, comm_ref,
             send_r, recv_r, send_l, recv_l):
        my_i = lax.axis_index("i")
        right_d = meta_ref[0]
        left_d = meta_ref[1]

        barrier_sem = pltpu.get_barrier_semaphore()
        for nbr in (left_d, right_d):
            pl.semaphore_signal(
                barrier_sem, inc=1,
                device_id=(nbr,), device_id_type=pl.DeviceIdType.MESH,
            )
        pl.semaphore_wait(barrier_sem, 2)

        sends = []

        def send(src, origin, half, dev, sem_arr, rsem_arr, hop):
            rdma = pltpu.make_async_remote_copy(
                src_ref=src,
                dst_ref=comm_ref.at[origin * N_SUB + half],
                send_sem=sem_arr.at[hop * N_SUB + half],
                recv_sem=rsem_arr.at[hop * N_SUB + half],
                device_id=(dev,),
                device_id_type=pl.DeviceIdType.MESH,
            )
            rdma.start()
            sends.append(rdma)

        def wait_recv(origin, half, dev, sem_arr, rsem_arr, hop):
            slot = origin * N_SUB + half
            pltpu.make_async_remote_copy(
                src_ref=comm_ref.at[slot],
                dst_ref=comm_ref.at[slot],
                send_sem=sem_arr.at[hop * N_SUB + half],
                recv_sem=rsem_arr.at[hop * N_SUB + half],
                device_id=(dev,),
                device_id_type=pl.DeviceIdType.MESH,
            ).wait_recv()

        def gemm(origin):
            out_ref[pl.ds(origin * m_per, m_per), :] = jnp.dot(
                comm_ref[pl.ds(origin * N_SUB, N_SUB)].reshape(m_per, k),
                w_ref[:, :],
                preferred_element_type=jnp.float32,
            )

        for s in range(N_SUB):
            half_src = x_ref.at[pl.ds(s * m_half, m_half), :]
            send(half_src, my_i, s, right_d, send_r, recv_r, 0)
            send(half_src, my_i, s, left_d, send_l, recv_l, 0)
        out_ref[pl.ds(my_i * m_per, m_per), :] = jnp.dot(
            x_ref[:, :], w_ref[:, :], preferred_element_type=jnp.float32
        )

        for h in range(1, AP + 1):
            r_o = meta_ref[2 + (h - 1)]
            l_o = meta_ref[2 + AP + (h - 1)]
            for s in range(N_SUB):
                if h < AP or s < HALF:
                    wait_recv(r_o, s, left_d, send_r, recv_r, h - 1)
                    if h + 1 < AP or (h + 1 == AP and s < HALF):
                        send(comm_ref.at[r_o * N_SUB + s], r_o, s,
                             right_d, send_r, recv_r, h)
                if h < AP or s >= HALF:
                    wait_recv(l_o, s, right_d, send_l, recv_l, h - 1)
                    if h + 1 < AP or (h + 1 == AP and s >= HALF):
                        send(comm_ref.at[l_o * N_SUB + s], l_o, s,
                             left_d, send_l, recv_l, h)
            gemm(r_o)
            if h < AP:
                gemm(l_o)

        for rdma in sends:
            rdma.wait_send()

    out_shape = jax.ShapeDtypeStruct((N_DEV * m_per, n_per), jnp.float32)
    return pl.pallas_call(
        body,
        out_shape=out_shape,
        in_specs=[
            pl.BlockSpec(memory_space=pltpu.SMEM),
            pl.BlockSpec(memory_space=pltpu.VMEM),
            pl.BlockSpec(memory_space=pltpu.VMEM),
        ],
        out_specs=pl.BlockSpec(memory_space=pltpu.VMEM),
        scratch_shapes=[
            pltpu.VMEM((N_DEV * N_SUB, m_half, k), x.dtype),
            pltpu.SemaphoreType.DMA((AP * N_SUB,)),
            pltpu.SemaphoreType.DMA((AP * N_SUB,)),
            pltpu.SemaphoreType.DMA((AP * N_SUB,)),
            pltpu.SemaphoreType.DMA((AP * N_SUB,)),
        ],
        compiler_params=pltpu.CompilerParams(collective_id=0),
    )(meta, x, w_mat)


# device time: 189735 ns/iter; 2.2586x vs baseline; 1.0111x over previous
import jax
import jax.numpy as jnp
from jax import lax
from jax.experimental import pallas as pl
from jax.experimental.pallas import tpu as pltpu

N_DEV = 32
AP = N_DEV // 2
N_SUB = 2
HALF = N_SUB // 2

RING = [0, 1, 9, 8, 16, 17, 25, 24, 27, 26, 18, 19, 11, 10, 13, 12,
        20, 21, 29, 28, 31, 30, 22, 23, 15, 14, 6, 7, 4, 5, 2, 3]
POS = [0] * N_DEV
for _p, _m in enumerate(RING):
    POS[_m] = _p


def kernel(x, w_mat):
    m_per, k = x.shape
    k2, n_per = w_mat.shape
    assert k == k2
    m_half = m_per // N_SUB

    ring3 = jnp.array(RING * 3, dtype=jnp.int32)
    pos_t = jnp.array(POS, dtype=jnp.int32)

    def body(ring_ref, pos_ref, x_ref, w_ref, out_ref,
             xb_ref, wb_ref, comm_ref, send_r, recv_r, send_l, recv_l):
        my_i = lax.axis_index("i")
        pos = pos_ref[my_i] + N_DEV
        right_d = ring_ref[pos + 1]
        left_d = ring_ref[pos - 1]

        barrier_sem = pltpu.get_barrier_semaphore()
        for nbr in (left_d, right_d):
            pl.semaphore_signal(
                barrier_sem, inc=1,
                device_id=(nbr,), device_id_type=pl.DeviceIdType.MESH,
            )
        pl.semaphore_wait(barrier_sem, 2)

        xb_ref[:, :] = x_ref[:, :].astype(jnp.bfloat16)
        wb_ref[:, :] = w_ref[:, :].astype(jnp.bfloat16)

        sends = []

        def send(src, origin, half, dev, sem_arr, rsem_arr, hop):
            rdma = pltpu.make_async_remote_copy(
                src_ref=src,
                dst_ref=comm_ref.at[origin * N_SUB + half],
                send_sem=sem_arr.at[hop * N_SUB + half],
                recv_sem=rsem_arr.at[hop * N_SUB + half],
                device_id=(dev,),
                device_id_type=pl.DeviceIdType.MESH,
            )
            rdma.start()
            sends.append(rdma)

        def wait_recv(origin, half, dev, sem_arr, rsem_arr, hop):
            slot = origin * N_SUB + half
            pltpu.make_async_remote_copy(
                src_ref=comm_ref.at[slot],
                dst_ref=comm_ref.at[slot],
                send_sem=sem_arr.at[hop * N_SUB + half],
                recv_sem=rsem_arr.at[hop * N_SUB + half],
                device_id=(dev,),
                device_id_type=pl.DeviceIdType.MESH,
            ).wait_recv()

        def gemm(origin):
            out_ref[pl.ds(origin * m_per, m_per), :] = jnp.dot(
                comm_ref[pl.ds(origin * N_SUB, N_SUB)].reshape(m_per, k),
                wb_ref[:, :],
                preferred_element_type=jnp.float32,
            )

        for s in range(N_SUB):
            half_src = xb_ref.at[pl.ds(s * m_half, m_half), :]
            send(half_src, my_i, s, right_d, send_r, recv_r, 0)
            send(half_src, my_i, s, left_d, send_l, recv_l, 0)
        out_ref[pl.ds(my_i * m_per, m_per), :] = jnp.dot(
            xb_ref[:, :], wb_ref[:, :], preferred_element_type=jnp.float32
        )

        for h in range(1, AP + 1):
            r_o = ring_ref[pos - h]
            l_o = ring_ref[pos + h]
            for s in range(N_SUB):
                if h < AP or s < HALF:
                    wait_recv(r_o, s, left_d, send_r, recv_r, h - 1)
                    if h + 1 < AP or (h + 1 == AP and s < HALF):
                        send(comm_ref.at[r_o * N_SUB + s], r_o, s,
                             right_d, send_r, recv_r, h)
                if h < AP or s >= HALF:
                    wait_recv(l_o, s, right_d, send_l, recv_l, h - 1)
                    if h + 1 < AP or (h + 1 == AP and s >= HALF):
                        send(comm_ref.at[l_o * N_SUB + s], l_o, s,
                             left_d, send_l, recv_l, h)
            gemm(r_o)
            if h < AP:
                gemm(l_o)

        for rdma in sends:
            rdma.wait_send()

    out_shape = jax.ShapeDtypeStruct((N_DEV * m_per, n_per), jnp.float32)
    return pl.pallas_call(
        body,
        out_shape=out_shape,
        in_specs=[
            pl.BlockSpec(memory_space=pltpu.SMEM),
            pl.BlockSpec(memory_space=pltpu.SMEM),
            pl.BlockSpec(memory_space=pltpu.VMEM),
            pl.BlockSpec(memory_space=pltpu.VMEM),
        ],
        out_specs=pl.BlockSpec(memory_space=pltpu.VMEM),
        scratch_shapes=[
            pltpu.VMEM((m_per, k), jnp.bfloat16),
            pltpu.VMEM((k, n_per), jnp.bfloat16),
            pltpu.VMEM((N_DEV * N_SUB, m_half, k), jnp.bfloat16),
            pltpu.SemaphoreType.DMA((AP * N_SUB,)),
            pltpu.SemaphoreType.DMA((AP * N_SUB,)),
            pltpu.SemaphoreType.DMA((AP * N_SUB,)),
            pltpu.SemaphoreType.DMA((AP * N_SUB,)),
        ],
        compiler_params=pltpu.CompilerParams(
            collective_id=0, vmem_limit_bytes=50 * 1024 * 1024
        ),
    )(ring3, pos_t, x, w_mat)
